# baseline (speedup 1.0000x reference)
import jax
import jax.numpy as jnp
from jax import lax
from jax.experimental import pallas as pl
from jax.experimental.pallas import tpu as pltpu
from jax.experimental.pallas import tpu_sc as plsc

VOCAB = 1000000
EMBED_DIM = 64
BATCH = 4096
SEQ_LEN = 200

NC = 2
NS = 16
NW = NC * NS

N = BATCH * SEQ_LEN
NB = N // NW
C = 512
G = NB // C


def _emb_body(table_hbm, text_hbm, out_hbm, idx_v, rows0, rows1,
              gsem0, gsem1, wsem0, wsem1):
    wid = lax.axis_index("s") * NC + lax.axis_index("c")

    pltpu.sync_copy(text_hbm.at[wid], idx_v)

    def idx_slice(g):
        return idx_v.at[pl.ds(g * C, C)]

    rows = (rows0, rows1)
    gsems = (gsem0, gsem1)
    wsems = (wsem0, wsem1)

    def start_gather(g, b):
        return pltpu.async_copy(table_hbm.at[idx_slice(g)], rows[b], gsems[b])

    def wait_gather(g, b):
        pltpu.make_async_copy(table_hbm.at[idx_slice(g)], rows[b],
                              gsems[b]).wait()

    def start_wb(g, b):
        return pltpu.async_copy(rows[b], out_hbm.at[wid, g], wsems[b])

    start_gather(0, 0)
    start_gather(1, 1)

    def pair(i, _):
        for b in range(2):
            g = 2 * i + b
            wait_gather(g, b)
            wb = start_wb(g, b)
            wb.wait()
            start_gather(g + 2, b)
        return 0

    lax.fori_loop(0, G // 2 - 1, pair, 0, unroll=False)

    wbs = []
    for b in range(2):
        g = G - 2 + b
        wait_gather(g, b)
        wbs.append(start_wb(g, b))
    for wb in wbs:
        wb.wait()


@jax.jit
def _embed(text_flat, table):
    mesh = plsc.VectorSubcoreMesh(core_axis_name="c", subcore_axis_name="s")
    k = pl.kernel(
        _emb_body,
        out_type=jax.ShapeDtypeStruct((NW, G, C, EMBED_DIM), jnp.float32),
        mesh=mesh,
        scratch_types=[
            pltpu.VMEM((NB,), jnp.int32),
            pltpu.VMEM((C, EMBED_DIM), jnp.float32),
            pltpu.VMEM((C, EMBED_DIM), jnp.float32),
            pltpu.SemaphoreType.DMA,
            pltpu.SemaphoreType.DMA,
            pltpu.SemaphoreType.DMA,
            pltpu.SemaphoreType.DMA,
        ],
        compiler_params=pltpu.CompilerParams(use_tc_tiling_on_sc=False),
    )
    return k(table, text_flat)


def kernel(text, table):
    text_flat = text.reshape(NW, NB)
    out = _embed(text_flat, table)
    return out.reshape(BATCH, SEQ_LEN, EMBED_DIM)
